# baseline (speedup 1.0000x reference)
import jax
import jax.numpy as jnp
from jax import lax
from jax.experimental import pallas as pl
from jax.experimental.pallas import tpu as pltpu
from jax.experimental.pallas import tpu_sc as plsc

_B, _S, _D = 16, 4096, 1024
_L = 16


def _body(x_hbm, m_hbm, out_hbm, mrow):
    wid = lax.axis_index("s") * 2 + lax.axis_index("c")

    @pl.when(wid < _B)
    def _():
        pltpu.sync_copy(m_hbm.at[wid], mrow)

        def step(i, a):
            return a + mrow[pl.ds(i * _L, _L)]

        acc = lax.fori_loop(0, _S // _L, step, jnp.zeros((_L,), jnp.int32))
        total = jnp.sum(acc)
        idx = jnp.where(total > 0, total - 1, _S - 1)
        pltpu.sync_copy(x_hbm.at[wid, pl.ds(idx, 1)], out_hbm.at[pl.ds(wid, 1)])


def kernel(x, mask):
    m32 = mask.astype(jnp.int32)
    mesh = plsc.VectorSubcoreMesh(core_axis_name="c", subcore_axis_name="s")
    run = pl.kernel(
        _body,
        mesh=mesh,
        out_type=jax.ShapeDtypeStruct((_B, _D), jnp.float32),
        scratch_types=[pltpu.VMEM((_S,), jnp.int32)],
        compiler_params=pltpu.CompilerParams(needs_layout_passes=False),
    )
    return run(x, m32)


# speedup vs baseline: 1.0283x; 1.0283x over previous
import jax
import jax.numpy as jnp
from jax import lax
from jax.experimental import pallas as pl
from jax.experimental.pallas import tpu as pltpu
from jax.experimental.pallas import tpu_sc as plsc

_B, _S, _D = 16, 4096, 1024
_L = 16
_W = _S // 4


def _body(x_hbm, m_hbm, out_hbm, mrow):
    wid = lax.axis_index("s")
    pltpu.sync_copy(m_hbm.at[wid], mrow)
    acc = mrow[pl.ds(0, _L)]
    for j in range(1, _W // _L):
        acc = acc + mrow[pl.ds(j * _L, _L)]
    t = (acc & 0x00FF00FF) + ((acc >> 8) & 0x00FF00FF)
    s = jnp.sum(t)
    total = (s & 0xFFFF) + (s >> 16)
    idx = jnp.where(total > 0, total - 1, _S - 1)
    pltpu.sync_copy(x_hbm.at[wid, pl.ds(idx, 1)], out_hbm.at[pl.ds(wid, 1)])


def kernel(x, mask):
    m32 = mask.view(jnp.int32)
    mesh = plsc.VectorSubcoreMesh(
        core_axis_name="c", subcore_axis_name="s", num_cores=1
    )
    run = pl.kernel(
        _body,
        mesh=mesh,
        out_type=jax.ShapeDtypeStruct((_B, _D), jnp.float32),
        scratch_types=[pltpu.VMEM((_W,), jnp.int32)],
        compiler_params=pltpu.CompilerParams(needs_layout_passes=False),
    )
    return run(x, m32)


# speedup vs baseline: 1.0883x; 1.0584x over previous
import jax
import jax.numpy as jnp
from jax import lax
from jax.experimental import pallas as pl
from jax.experimental.pallas import tpu as pltpu
from jax.experimental.pallas import tpu_sc as plsc

_B, _S, _D = 16, 4096, 1024
_L = 16
_UNROLL = 8


def _body(x_hbm, m_hbm, out_hbm, mrow):
    wid = lax.axis_index("s")
    pltpu.sync_copy(m_hbm.at[wid], mrow)

    def step(i, a):
        base = i * (_L * _UNROLL)
        for j in range(_UNROLL):
            a = a + mrow[pl.ds(base + j * _L, _L)]
        return a

    acc = lax.fori_loop(
        0, _S // (_L * _UNROLL), step, jnp.zeros((_L,), jnp.int32)
    )
    total = jnp.sum(acc)
    idx = jnp.where(total > 0, total - 1, _S - 1)
    pltpu.sync_copy(x_hbm.at[wid, pl.ds(idx, 1)], out_hbm.at[pl.ds(wid, 1)])


def kernel(x, mask):
    m32 = mask.astype(jnp.int32)
    mesh = plsc.VectorSubcoreMesh(
        core_axis_name="c", subcore_axis_name="s", num_cores=1
    )
    run = pl.kernel(
        _body,
        mesh=mesh,
        out_type=jax.ShapeDtypeStruct((_B, _D), jnp.float32),
        scratch_types=[pltpu.VMEM((_S,), jnp.int32)],
        compiler_params=pltpu.CompilerParams(needs_layout_passes=False),
    )
    return run(x, m32)


# speedup vs baseline: 1.0894x; 1.0011x over previous
import jax
import jax.numpy as jnp
from jax import lax
from jax.experimental import pallas as pl
from jax.experimental.pallas import tpu as pltpu
from jax.experimental.pallas import tpu_sc as plsc

_B, _S, _D = 16, 4096, 1024
_L = 16
_UNROLL = 8


def _body(x_hbm, m_hbm, out_hbm, mrow):
    wid = lax.axis_index("s")
    pltpu.sync_copy(m_hbm.at[wid], mrow)

    def step(i, a):
        base = i * (_L * _UNROLL)
        for j in range(_UNROLL):
            a = a + mrow[pl.ds(base + j * _L, _L)]
        return a

    acc = lax.fori_loop(
        0, _S // (_L * _UNROLL), step, jnp.zeros((_L,), jnp.int32)
    )
    total = jnp.sum(acc)
    idx = jnp.where(total > 0, total - 1, _S - 1)
    pltpu.sync_copy(x_hbm.at[wid, pl.ds(idx, 1)], out_hbm.at[pl.ds(wid, 1)])


def kernel(x, mask):
    m32 = mask.astype(jnp.int32)
    mesh = plsc.VectorSubcoreMesh(
        core_axis_name="c", subcore_axis_name="s", num_cores=1
    )
    run = pl.kernel(
        _body,
        mesh=mesh,
        out_type=jax.ShapeDtypeStruct((_B, _D), jnp.float32),
        scratch_types=[pltpu.VMEM((_S,), jnp.int32)],
        compiler_params=pltpu.CompilerParams(
            needs_layout_passes=False,
            disable_bounds_checks=True,
            disable_semaphore_checks=True,
            skip_device_barrier=True,
        ),
    )
    return run(x, m32)
